# baseline (speedup 1.0000x reference)
import functools

import jax
import jax.numpy as jnp
from jax import lax
from jax.experimental import pallas as pl
from jax.experimental.pallas import tpu as pltpu
from jax.experimental.pallas import tpu_sc as plsc


_SQ_BLOCK_ROWS = 2000


def _square_body(x_ref, o_ref):
    x = x_ref[...]
    o_ref[...] = x * x


def _square2d(p2d):
    rows, cols = p2d.shape
    grid = (rows // _SQ_BLOCK_ROWS,)
    return pl.pallas_call(
        _square_body,
        grid=grid,
        in_specs=[pl.BlockSpec((_SQ_BLOCK_ROWS, cols), lambda i: (i, 0))],
        out_specs=pl.BlockSpec((_SQ_BLOCK_ROWS, cols), lambda i: (i, 0)),
        out_shape=jax.ShapeDtypeStruct((rows, cols), jnp.float32),
    )(p2d)


def _make_sc_gather(n_rows, d, b):
    info = plsc.get_sparse_core_info()
    nc, ns = info.num_cores, info.num_subcores
    nw = nc * ns
    b_per_w = b // nw
    mesh = plsc.VectorSubcoreMesh(core_axis_name="c", subcore_axis_name="s")

    @functools.partial(
        pl.kernel,
        mesh=mesh,
        out_type=jax.ShapeDtypeStruct((b, d), jnp.float32),
        scratch_types=[
            pltpu.VMEM((b_per_w,), jnp.int32),
            pltpu.VMEM((b_per_w, d), jnp.float32),
            pltpu.SemaphoreType.DMA,
        ],
        compiler_params=pltpu.CompilerParams(use_tc_tiling_on_sc=False),
    )
    def gather_k(table_hbm, idx_hbm, out_hbm, idx_v, rows_v, sem):
        wid = lax.axis_index("s") * nc + lax.axis_index("c")
        base = wid * b_per_w
        pltpu.sync_copy(idx_hbm.at[pl.ds(base, b_per_w)], idx_v)
        pltpu.async_copy(table_hbm.at[idx_v], rows_v, sem).wait()
        pltpu.sync_copy(rows_v, out_hbm.at[pl.ds(base, b_per_w)])

    return gather_k


def kernel(indices, core_param):
    r1, n, r2 = core_param.shape
    b = indices.shape[0]
    table = core_param.reshape(r1 * n, r2)
    out = _make_sc_gather(r1 * n, r2, b)(table, indices.astype(jnp.int32))
    reg = _square2d(core_param.reshape(-1, 128)).reshape(r1, n, r2)
    return out.reshape(b, r1, r2), reg


# speedup vs baseline: 1.3148x; 1.3148x over previous
import functools

import jax
import jax.numpy as jnp
from jax import lax
from jax.experimental import pallas as pl
from jax.experimental.pallas import tpu as pltpu
from jax.experimental.pallas import tpu_sc as plsc

_CHUNK_ROWS = 1250


def _make_fused(n_rows, d, b):
    info = plsc.get_sparse_core_info()
    nc, ns = info.num_cores, info.num_subcores
    nw = nc * ns
    b_per_w = b // nw
    rows_per_w = n_rows // nw
    n_chunks = rows_per_w // _CHUNK_ROWS
    mesh = plsc.VectorSubcoreMesh(core_axis_name="c", subcore_axis_name="s")

    @functools.partial(
        pl.kernel,
        mesh=mesh,
        out_type=(
            jax.ShapeDtypeStruct((b, d), jnp.float32),
            jax.ShapeDtypeStruct((n_rows, d), jnp.float32),
        ),
        scratch_types=[
            pltpu.VMEM((b_per_w,), jnp.int32),
            pltpu.VMEM((b_per_w, d), jnp.float32),
            pltpu.VMEM((_CHUNK_ROWS, d), jnp.float32),
            pltpu.SemaphoreType.DMA,
        ],
        compiler_params=pltpu.CompilerParams(use_tc_tiling_on_sc=False),
    )
    def fused_k(tbl_hbm, idx_hbm, out_hbm, reg_hbm, idx_v, rows_v, buf, sem):
        wid = lax.axis_index("s") * nc + lax.axis_index("c")
        base = wid * b_per_w
        pltpu.sync_copy(idx_hbm.at[pl.ds(base, b_per_w)], idx_v)
        pltpu.async_copy(tbl_hbm.at[idx_v], rows_v, sem).wait()
        pltpu.sync_copy(rows_v, out_hbm.at[pl.ds(base, b_per_w)])

        row0 = wid * rows_per_w

        @pl.loop(0, n_chunks)
        def _chunk(ci):
            r = row0 + ci * _CHUNK_ROWS
            pltpu.sync_copy(tbl_hbm.at[pl.ds(r, _CHUNK_ROWS)], buf)

            @functools.partial(plsc.parallel_loop, 0, _CHUNK_ROWS, unroll=8)
            def _row(i):
                a = buf[i, pl.ds(0, 16)]
                buf[i, pl.ds(0, 16)] = a * a
                c = buf[i, pl.ds(16, 16)]
                buf[i, pl.ds(16, 16)] = c * c

            pltpu.sync_copy(buf, reg_hbm.at[pl.ds(r, _CHUNK_ROWS)])

    return fused_k


def kernel(indices, core_param):
    r1, n, r2 = core_param.shape
    b = indices.shape[0]
    table = core_param.reshape(r1 * n, r2)
    out2d, reg2d = _make_fused(r1 * n, r2, b)(table, indices.astype(jnp.int32))
    return out2d.reshape(b, r1, r2), reg2d.reshape(r1, n, r2)


# speedup vs baseline: 1.3349x; 1.0153x over previous
import functools

import jax
import jax.numpy as jnp
from jax import lax
from jax.experimental import pallas as pl
from jax.experimental.pallas import tpu as pltpu
from jax.experimental.pallas import tpu_sc as plsc

_CH = 625


def _make_fused(n_rows, d, b):
    info = plsc.get_sparse_core_info()
    nc, ns = info.num_cores, info.num_subcores
    nw = nc * ns
    b_per_w = b // nw
    rows_per_w = n_rows // nw
    n_chunks = rows_per_w // _CH
    mesh = plsc.VectorSubcoreMesh(core_axis_name="c", subcore_axis_name="s")

    @functools.partial(
        pl.kernel,
        mesh=mesh,
        out_type=(
            jax.ShapeDtypeStruct((b, 1, d), jnp.float32),
            jax.ShapeDtypeStruct((1, n_rows, d), jnp.float32),
        ),
        scratch_types=[
            pltpu.VMEM((b_per_w,), jnp.int32),
            pltpu.VMEM((b_per_w, 1, d), jnp.float32),
            pltpu.VMEM((_CH, d), jnp.float32),
            pltpu.VMEM((_CH, d), jnp.float32),
            pltpu.SemaphoreType.DMA,
            pltpu.SemaphoreType.DMA,
            pltpu.SemaphoreType.DMA,
        ],
        compiler_params=pltpu.CompilerParams(use_tc_tiling_on_sc=False),
    )
    def fused_k(p_hbm, idx_hbm, out_hbm, reg_hbm, idx_v, rows_v, buf_a,
                buf_b, sem_g, sem_a, sem_b):
        wid = lax.axis_index("s") * nc + lax.axis_index("c")
        tbl = p_hbm.at[0]
        reg = reg_hbm.at[0]

        base = wid * b_per_w
        pltpu.sync_copy(idx_hbm.at[pl.ds(base, b_per_w)], idx_v)
        pltpu.async_copy(tbl.at[idx_v], rows_v.at[:, 0], sem_g).wait()
        pltpu.sync_copy(rows_v, out_hbm.at[pl.ds(base, b_per_w)])

        row0 = wid * rows_per_w

        def start(ci, buf, sem):
            pltpu.async_copy(tbl.at[pl.ds(row0 + ci * _CH, _CH)], buf, sem)

        def wait(ci, buf, sem):
            pltpu.make_async_copy(
                tbl.at[pl.ds(row0 + ci * _CH, _CH)], buf, sem).wait()

        def square_and_flush(ci, buf):
            @functools.partial(plsc.parallel_loop, 0, _CH, unroll=8)
            def _row(i):
                a = buf[i, pl.ds(0, 16)]
                buf[i, pl.ds(0, 16)] = a * a
                c = buf[i, pl.ds(16, 16)]
                buf[i, pl.ds(16, 16)] = c * c

            pltpu.sync_copy(buf, reg.at[pl.ds(row0 + ci * _CH, _CH)])

        start(0, buf_a, sem_a)

        @pl.loop(0, n_chunks, step=2)
        def _chunks(c):
            start(c + 1, buf_b, sem_b)
            wait(c, buf_a, sem_a)
            square_and_flush(c, buf_a)

            @pl.when(c + 2 < n_chunks)
            def _():
                start(c + 2, buf_a, sem_a)

            wait(c + 1, buf_b, sem_b)
            square_and_flush(c + 1, buf_b)

    return fused_k


def kernel(indices, core_param):
    r1, n, r2 = core_param.shape
    b = indices.shape[0]
    out, reg = _make_fused(n, r2, b)(core_param, indices.astype(jnp.int32))
    return out, reg
